# baseline (device time: 54116 ns/iter reference)
import jax
import jax.numpy as jnp
from jax import lax
from jax.experimental import pallas as pl
from jax.experimental.pallas import tpu as pltpu

N_DEV = 4


def kernel(x):
    m, n = x.shape

    def body(x_ref, out_ref, totals_ref, send_sems, recv_sems):
        my = lax.axis_index("i")

        barrier_sem = pltpu.get_barrier_semaphore()
        for off in range(1, N_DEV):
            pl.semaphore_signal(
                barrier_sem,
                inc=1,
                device_id=((my + off) % N_DEV,),
                device_id_type=pl.DeviceIdType.MESH,
            )
        pl.semaphore_wait(barrier_sem, N_DEV - 1)

        x = x_ref[:, :]

        r = x
        while r.shape[0] > 1:
            h = r.shape[0] // 2
            r = r[:h, :] * r[h:, :]
        totals_ref[pl.ds(my, 1), :] = r

        sends = []
        for k in range(N_DEV - 1):
            rdma = pltpu.make_async_remote_copy(
                src_ref=totals_ref.at[pl.ds(my, 1)],
                dst_ref=totals_ref.at[pl.ds(my, 1)],
                send_sem=send_sems.at[k],
                recv_sem=recv_sems.at[k],
                device_id=((my + k + 1) % N_DEV,),
                device_id_type=pl.DeviceIdType.MESH,
            )
            rdma.start()
            sends.append(rdma)

        row = lax.broadcasted_iota(jnp.int32, (m, n), 0)
        s = 1
        while s < m:
            shifted = pltpu.roll(x, s, 0)
            x = x * jnp.where(row < s, jnp.float32(1.0), shifted)
            s *= 2

        for k in range(N_DEV - 1):
            src_row = (my - 1 - k) % N_DEV
            recv = pltpu.make_async_remote_copy(
                src_ref=totals_ref.at[pl.ds(src_row, 1)],
                dst_ref=totals_ref.at[pl.ds(src_row, 1)],
                send_sem=send_sems.at[k],
                recv_sem=recv_sems.at[k],
                device_id=(my,),
                device_id_type=pl.DeviceIdType.MESH,
            )
            recv.wait_recv()
        for rdma in sends:
            rdma.wait_send()

        totals = totals_ref[:, :]
        rid = lax.broadcasted_iota(jnp.int32, (N_DEV, n), 0)
        factors = jnp.where(rid < my, totals, jnp.ones_like(totals))
        prefix = factors[0] * factors[1] * factors[2] * factors[3]
        out_ref[:, :] = x * prefix[None, :]

    return pl.pallas_call(
        body,
        out_shape=jax.ShapeDtypeStruct((m, n), jnp.float32),
        in_specs=[pl.BlockSpec(memory_space=pltpu.VMEM)],
        out_specs=pl.BlockSpec(memory_space=pltpu.VMEM),
        scratch_shapes=[
            pltpu.VMEM((N_DEV, n), jnp.float32),
            pltpu.SemaphoreType.DMA((N_DEV - 1,)),
            pltpu.SemaphoreType.DMA((N_DEV - 1,)),
        ],
        compiler_params=pltpu.CompilerParams(
            collective_id=0, vmem_limit_bytes=100 * 1024 * 1024
        ),
    )(x)


# device time: 53549 ns/iter; 1.0106x vs baseline; 1.0106x over previous
import jax
import jax.numpy as jnp
from jax import lax
from jax.experimental import pallas as pl
from jax.experimental.pallas import tpu as pltpu

N_DEV = 4


def kernel(x):
    m, n = x.shape

    def body(x_ref, out_ref, totals_ref, send_sems, recv_sems):
        my = lax.axis_index("i")

        barrier_sem = pltpu.get_barrier_semaphore()
        for off in range(1, N_DEV):
            pl.semaphore_signal(
                barrier_sem,
                inc=1,
                device_id=((my + off) % N_DEV,),
                device_id_type=pl.DeviceIdType.MESH,
            )
        pl.semaphore_wait(barrier_sem, N_DEV - 1)

        x = x_ref[:, :]

        row = lax.broadcasted_iota(jnp.int32, (m, n), 0)
        s = 1
        while s < m:
            shifted = pltpu.roll(x, s, 0)
            x = x * jnp.where(row < s, jnp.float32(1.0), shifted)
            s *= 2
        totals_ref[pl.ds(my, 1), :] = x[m - 1 : m, :]

        sends = []
        for k in range(N_DEV - 1):
            rdma = pltpu.make_async_remote_copy(
                src_ref=totals_ref.at[pl.ds(my, 1)],
                dst_ref=totals_ref.at[pl.ds(my, 1)],
                send_sem=send_sems.at[k],
                recv_sem=recv_sems.at[k],
                device_id=((my + k + 1) % N_DEV,),
                device_id_type=pl.DeviceIdType.MESH,
            )
            rdma.start()
            sends.append(rdma)

        for k in range(N_DEV - 1):
            src_row = (my - 1 - k) % N_DEV
            recv = pltpu.make_async_remote_copy(
                src_ref=totals_ref.at[pl.ds(src_row, 1)],
                dst_ref=totals_ref.at[pl.ds(src_row, 1)],
                send_sem=send_sems.at[k],
                recv_sem=recv_sems.at[k],
                device_id=(my,),
                device_id_type=pl.DeviceIdType.MESH,
            )
            recv.wait_recv()
        for rdma in sends:
            rdma.wait_send()

        totals = totals_ref[:, :]
        rid = lax.broadcasted_iota(jnp.int32, (N_DEV, n), 0)
        factors = jnp.where(rid < my, totals, jnp.ones_like(totals))
        prefix = factors[0] * factors[1] * factors[2] * factors[3]
        out_ref[:, :] = x * prefix[None, :]

    return pl.pallas_call(
        body,
        out_shape=jax.ShapeDtypeStruct((m, n), jnp.float32),
        in_specs=[pl.BlockSpec(memory_space=pltpu.VMEM)],
        out_specs=pl.BlockSpec(memory_space=pltpu.VMEM),
        scratch_shapes=[
            pltpu.VMEM((N_DEV, n), jnp.float32),
            pltpu.SemaphoreType.DMA((N_DEV - 1,)),
            pltpu.SemaphoreType.DMA((N_DEV - 1,)),
        ],
        compiler_params=pltpu.CompilerParams(
            collective_id=0, vmem_limit_bytes=100 * 1024 * 1024
        ),
    )(x)


# device time: 42009 ns/iter; 1.2882x vs baseline; 1.2747x over previous
import jax
import jax.numpy as jnp
from jax import lax
from jax.experimental import pallas as pl
from jax.experimental.pallas import tpu as pltpu

N_DEV = 4


def kernel(x):
    m, n = x.shape

    def body(x_ref, out_ref, totals_ref, send_sems, recv_sems):
        my = lax.axis_index("i")

        barrier_sem = pltpu.get_barrier_semaphore()
        for off in range(1, N_DEV):
            pl.semaphore_signal(
                barrier_sem,
                inc=1,
                device_id=((my + off) % N_DEV,),
                device_id_type=pl.DeviceIdType.MESH,
            )
        pl.semaphore_wait(barrier_sem, N_DEV - 1)

        x = x_ref[:, :]

        row = lax.broadcasted_iota(jnp.int32, (m, n), 0)
        s = 1
        while s < m:
            shifted = pltpu.roll(x, s, 0)
            x = x * jnp.where(row < s, jnp.float32(1.0), shifted)
            s *= 2
        out_ref[:, :] = x
        totals_ref[pl.ds(my, 1), :] = x[m - 1 : m, :]

        sends = []
        for k in range(N_DEV - 1):
            rdma = pltpu.make_async_remote_copy(
                src_ref=totals_ref.at[pl.ds(my, 1)],
                dst_ref=totals_ref.at[pl.ds(my, 1)],
                send_sem=send_sems.at[k],
                recv_sem=recv_sems.at[k],
                device_id=((my + k + 1) % N_DEV,),
                device_id_type=pl.DeviceIdType.MESH,
            )
            rdma.start()
            sends.append(rdma)

        for k in range(N_DEV - 1):
            src_row = (my - 1 - k) % N_DEV
            recv = pltpu.make_async_remote_copy(
                src_ref=totals_ref.at[pl.ds(src_row, 1)],
                dst_ref=totals_ref.at[pl.ds(src_row, 1)],
                send_sem=send_sems.at[k],
                recv_sem=recv_sems.at[k],
                device_id=(my,),
                device_id_type=pl.DeviceIdType.MESH,
            )
            recv.wait_recv()
        for rdma in sends:
            rdma.wait_send()

        totals = totals_ref[:, :]
        rid = lax.broadcasted_iota(jnp.int32, (N_DEV, n), 0)
        factors = jnp.where(rid < my, totals, jnp.ones_like(totals))
        prefix = factors[0] * factors[1] * factors[2] * factors[3]
        out_ref[:, :] = out_ref[:, :] * prefix[None, :]

    return pl.pallas_call(
        body,
        out_shape=jax.ShapeDtypeStruct((m, n), jnp.float32),
        in_specs=[pl.BlockSpec(memory_space=pltpu.VMEM)],
        out_specs=pl.BlockSpec(memory_space=pltpu.VMEM),
        scratch_shapes=[
            pltpu.VMEM((N_DEV, n), jnp.float32),
            pltpu.SemaphoreType.DMA((N_DEV - 1,)),
            pltpu.SemaphoreType.DMA((N_DEV - 1,)),
        ],
        compiler_params=pltpu.CompilerParams(
            collective_id=0, vmem_limit_bytes=100 * 1024 * 1024
        ),
    )(x)


# device time: 29250 ns/iter; 1.8501x vs baseline; 1.4362x over previous
import jax
import jax.numpy as jnp
from jax import lax
from jax.experimental import pallas as pl
from jax.experimental.pallas import tpu as pltpu

N_DEV = 4


def kernel(x):
    m, n = x.shape

    def body(x_ref, out_ref, totals_ref, send_sems, recv_sems):
        my = lax.axis_index("i")

        barrier_sem = pltpu.get_barrier_semaphore()
        for off in range(1, N_DEV):
            pl.semaphore_signal(
                barrier_sem,
                inc=1,
                device_id=((my + off) % N_DEV,),
                device_id_type=pl.DeviceIdType.MESH,
            )
        pl.semaphore_wait(barrier_sem, N_DEV - 1)

        x = x_ref[:, :]

        row = lax.broadcasted_iota(jnp.int32, (m, n), 0)
        s = 1
        while s < m:
            shifted = pltpu.roll(x, s, 0)
            x = x * jnp.where(row < s, jnp.float32(1.0), shifted)
            s *= 2
        out_ref[:, :] = x
        totals_ref[pl.ds(my, 1), :] = x[m - 1 : m, :]

        sends = []
        for k in range(N_DEV - 1):
            rdma = pltpu.make_async_remote_copy(
                src_ref=totals_ref.at[pl.ds(my, 1)],
                dst_ref=totals_ref.at[pl.ds(my, 1)],
                send_sem=send_sems.at[k],
                recv_sem=recv_sems.at[k],
                device_id=((my + k + 1) % N_DEV,),
                device_id_type=pl.DeviceIdType.MESH,
            )
            rdma.start()
            sends.append(rdma)

        for k in range(N_DEV - 1):
            src_row = (my - 1 - k) % N_DEV
            recv = pltpu.make_async_remote_copy(
                src_ref=totals_ref.at[pl.ds(src_row, 1)],
                dst_ref=totals_ref.at[pl.ds(src_row, 1)],
                send_sem=send_sems.at[k],
                recv_sem=recv_sems.at[k],
                device_id=(my,),
                device_id_type=pl.DeviceIdType.MESH,
            )
            recv.wait_recv()
        for rdma in sends:
            rdma.wait_send()

        totals = totals_ref[:, :]
        rid = lax.broadcasted_iota(jnp.int32, (N_DEV, n), 0)
        factors = jnp.where(rid < my, totals, jnp.ones_like(totals))
        prefix = factors[0] * factors[1] * factors[2] * factors[3]
        out_ref[:, :] = out_ref[:, :] * prefix[None, :]

    return pl.pallas_call(
        body,
        out_shape=jax.ShapeDtypeStruct((m, n), jnp.float32),
        in_specs=[pl.BlockSpec(memory_space=pltpu.VMEM)],
        out_specs=pl.BlockSpec(memory_space=pltpu.VMEM),
        scratch_shapes=[
            pltpu.VMEM((N_DEV, n), jnp.float32),
            pltpu.SemaphoreType.DMA((N_DEV - 1,)),
            pltpu.SemaphoreType.DMA((N_DEV - 1,)),
        ],
        compiler_params=pltpu.CompilerParams(collective_id=0),
    )(x)
